# Initial kernel scaffold; baseline (speedup 1.0000x reference)
import jax
import jax.numpy as jnp
from jax.experimental import pallas as pl


def kernel(x, edge_index, W_lin, b_lin, W1w, W1b, W2w, W2b, W3w, W3b, Wcw, Wcb):
    raise NotImplementedError("write your pallas kernel here")


# speedup vs baseline: 2.7800x; 2.7800x over previous
import functools

import jax
import jax.numpy as jnp
from jax import lax
from jax.experimental import pallas as pl
from jax.experimental.pallas import tpu as pltpu
from jax.experimental.pallas import tpu_sc as plsc

N = 10000
E = 320000
D = 128
NP = 10240
NC = 2
NS = 16
NW = NC * NS
K = 128
CH = 80
EPT = E // NW
RPT = NP // NS
RCH = RPT // K


def _scatter_body(table, src_idx, dst_idx, zrow, out, src_v, dst_v, rows_v,
                  acc_sh, sem):
    c = lax.axis_index("c")
    s = lax.axis_index("s")
    wid = c * NS + s

    pltpu.sync_copy(src_idx.at[wid], src_v)
    pltpu.sync_copy(dst_idx.at[wid], dst_v)
    for r in range(RCH):
        off = pl.multiple_of(s * RPT + r * K, K)
        pltpu.sync_copy(zrow, acc_sh.at[pl.ds(off, K)])
    plsc.subcore_barrier()

    def chunk(j, carry):
        pltpu.async_copy(table.at[src_v.at[j]], rows_v, sem).wait()
        pltpu.sync_copy(rows_v, acc_sh.at[dst_v.at[j]], add=True)
        return carry

    lax.fori_loop(0, CH, chunk, 0)
    plsc.subcore_barrier()

    for r in range(RCH):
        off = pl.multiple_of(s * RPT + r * K, K)
        pltpu.sync_copy(acc_sh.at[pl.ds(off, K)], rows_v)
        pltpu.sync_copy(rows_v, out.at[c, pl.ds(off, K)])


def _scatter2_body(table_b, table_a, edges_idx, vert_idx, zrow, out,
                   e_v, v_v, rows_v, acc_sh, sem):
    c = lax.axis_index("c")
    s = lax.axis_index("s")
    wid = c * NS + s

    pltpu.sync_copy(edges_idx.at[wid], e_v)
    pltpu.sync_copy(vert_idx.at[wid], v_v)
    for r in range(RCH):
        off = pl.multiple_of(s * RPT + r * K, K)
        pltpu.sync_copy(zrow, acc_sh.at[pl.ds(off, K)])
    plsc.subcore_barrier()

    def chunk(j, carry):
        pltpu.async_copy(table_b.at[e_v.at[j]], rows_v, sem).wait()
        pltpu.async_copy(table_a.at[v_v.at[j]], rows_v, sem, add=True).wait()
        pltpu.sync_copy(rows_v, acc_sh.at[v_v.at[j]], add=True)
        return carry

    lax.fori_loop(0, CH, chunk, 0)
    plsc.subcore_barrier()

    for r in range(RCH):
        off = pl.multiple_of(s * RPT + r * K, K)
        pltpu.sync_copy(acc_sh.at[pl.ds(off, K)], rows_v)
        pltpu.sync_copy(rows_v, out.at[c, pl.ds(off, K)])


@functools.lru_cache(maxsize=None)
def _make_scatter(two_tables):
    mesh = plsc.VectorSubcoreMesh(core_axis_name="c", subcore_axis_name="s",
                                  num_cores=NC, num_subcores=NS)
    return pl.kernel(
        _scatter2_body if two_tables else _scatter_body,
        out_type=jax.ShapeDtypeStruct((NC, NP, D), jnp.float32),
        mesh=mesh,
        scratch_types=[
            pltpu.VMEM((CH, K), jnp.int32),
            pltpu.VMEM((CH, K), jnp.int32),
            pltpu.VMEM((K, D), jnp.float32),
            pltpu.VMEM_SHARED((NP, D), jnp.float32),
            pltpu.SemaphoreType.DMA,
        ],
    )


def _t0_body(x_ref, wl_ref, bl_ref, w1_ref, b1_ref, h_ref, g_ref):
    h = jnp.maximum(
        jnp.dot(x_ref[...], wl_ref[...], preferred_element_type=jnp.float32)
        + bl_ref[...], 0.0)
    h_ref[...] = h
    g_ref[...] = (jnp.dot(h, w1_ref[...], preferred_element_type=jnp.float32)
                  + b1_ref[...])


def _t1_body(p_ref, h_ref, w2a_ref, w2b_ref, b2_ref, a_ref, b_ref):
    xe = p_ref[0] + p_ref[1]
    a_ref[...] = (jnp.dot(h_ref[...], w2a_ref[...],
                          preferred_element_type=jnp.float32) + b2_ref[...])
    b_ref[...] = jnp.dot(xe, w2b_ref[...], preferred_element_type=jnp.float32)


def _mid_body(q_ref, h0_ref, w3_ref, b3_ref, w1_ref, b1_ref, h2_ref, g2_ref):
    xv = q_ref[0] + q_ref[1]
    u = 0.5 * xv + 0.5 * h0_ref[...]
    h2 = jnp.maximum(
        jnp.dot(u, w3_ref[...], preferred_element_type=jnp.float32)
        + b3_ref[...], 0.0)
    h2_ref[...] = h2
    g2_ref[...] = (jnp.dot(h2, w1_ref[...], preferred_element_type=jnp.float32)
                   + b1_ref[...])


def _final_body(q_ref, h0_ref, w3_ref, b3_ref, wc_ref, bc_ref, out_ref):
    xv = q_ref[0] + q_ref[1]
    u = 0.5 * xv + 0.5 * h0_ref[...]
    h3 = jnp.maximum(
        jnp.dot(u, w3_ref[...], preferred_element_type=jnp.float32)
        + b3_ref[...], 0.0)
    out_ref[...] = (jnp.dot(h3[:N], wc_ref[...],
                            preferred_element_type=jnp.float32) + bc_ref[...])


def _tc(body, out_shapes, *args):
    return pl.pallas_call(body, out_shape=out_shapes)(*args)


def kernel(x, edge_index, W_lin, b_lin, W1w, W1b, W2w, W2b, W3w, W3b, Wcw, Wcb):
    f32 = jnp.float32
    pad = jnp.full((NW, CH * K - EPT), N, jnp.int32)
    vertex = jnp.concatenate(
        [edge_index[0].reshape(NW, EPT), pad], axis=1).reshape(NW, CH, K)
    edges = jnp.concatenate(
        [edge_index[1].reshape(NW, EPT), pad], axis=1).reshape(NW, CH, K)
    xp = jnp.pad(x, ((0, NP - N), (0, 0)))
    zrow = jnp.zeros((K, D), f32)
    W2a, W2bb = W2w[:D], W2w[D:]
    bl = b_lin.reshape(1, D)
    b1 = W1b.reshape(1, D)
    b2 = W2b.reshape(1, D)
    b3 = W3b.reshape(1, D)
    bc = Wcb.reshape(1, -1)

    nd = jax.ShapeDtypeStruct((NP, D), f32)
    scat = _make_scatter(False)
    scat2 = _make_scatter(True)

    h0, g1 = _tc(_t0_body, (nd, nd), xp, W_lin, bl, W1w, b1)

    p1 = scat(g1, vertex, edges, zrow)
    a1, bt1 = _tc(_t1_body, (nd, nd), p1, h0, W2a, W2bb, b2)
    q1 = scat2(bt1, a1, edges, vertex, zrow)
    h2, g2 = _tc(_mid_body, (nd, nd), q1, h0, W3w, b3, W1w, b1)

    p2 = scat(g2, vertex, edges, zrow)
    a2, bt2 = _tc(_t1_body, (nd, nd), p2, h2, W2a, W2bb, b2)
    q2 = scat2(bt2, a2, edges, vertex, zrow)
    out = _tc(_final_body, jax.ShapeDtypeStruct((N, Wcw.shape[1]), f32),
              q2, h0, W3w, b3, Wcw, bc)
    return out
